# baseline (speedup 1.0000x reference)
import jax
import jax.numpy as jnp
from jax import lax
from jax.experimental import pallas as pl
from jax.experimental.pallas import tpu as pltpu
from jax.experimental.pallas import tpu_sc as plsc

_VOCAB = 100000
_D = 128
_L = 2048
_B = 32
_NC = 2
_NS = 16
_NW = _NC * _NS
_NHALF = 2
_BW = _B // _NHALF
_NCOL = _NW // _NHALF
_LW = _L // _NCOL
_LANES = 16
_CW = _LW // 2
_NCHUNK = 2 * _BW
_NBUF = 4
_GROUPS = _NCHUNK // _NBUF


def _emb_body(seq_hbm, table_hbm, pos_hbm, out_hbm, idx_v, pos_v, tok_v,
              gsems, osems, psem):
    wid = lax.axis_index("s") * _NC + lax.axis_index("c")
    half = wid // _NCOL
    colblk = wid % _NCOL
    b0 = half * _BW
    l0 = colblk * _LW

    pltpu.sync_copy(seq_hbm.at[pl.ds(b0, _BW), pl.ds(l0, _LW)], idx_v)
    pos_copy = pltpu.async_copy(
        pos_hbm.at[pl.ds(l0 * _D, _LW * _D)], pos_v, psem)

    def gstart(ck, buf, hk):
        lb = lax.div(ck, 2) if not isinstance(ck, int) else ck // 2
        pltpu.async_copy(
            table_hbm.at[idx_v.at[lb, pl.ds(hk * _CW, _CW)]],
            tok_v.at[buf], gsems[buf])

    def gwait(buf):
        pltpu.make_async_copy(
            table_hbm.at[pl.ds(0, _CW)], tok_v.at[buf], gsems[buf]).wait()

    def ostart(ck, buf, hk):
        lb = lax.div(ck, 2) if not isinstance(ck, int) else ck // 2
        pltpu.async_copy(
            tok_v.at[buf],
            out_hbm.at[pl.ds((b0 + lb) * _L + l0 + hk * _CW, _CW)],
            osems[buf])

    def owait(buf):
        pltpu.make_async_copy(
            tok_v.at[buf], out_hbm.at[pl.ds(0, _CW)], osems[buf]).wait()

    def add_pos(buf, hk):
        @pl.loop(0, _CW, unroll=1)
        def _(r):
            for c in range(_D // _LANES):
                plsc.addupdate(
                    tok_v.at[buf, r, pl.ds(c * _LANES, _LANES)],
                    pos_v[pl.ds((hk * _CW + r) * _D + c * _LANES, _LANES)],
                )

    for ck in range(_NBUF):
        gstart(ck, ck, ck % 2)
    pos_copy.wait()

    def group(i, carry):
        for j in range(_NBUF):
            ck = _NBUF * i + j
            hk = j % 2
            s = ck - 2
            sbuf = (j - 2) % _NBUF

            @pl.when(jnp.logical_and(s >= 0, s + _NBUF < _NCHUNK))
            def _():
                owait(sbuf)
                gstart(s + _NBUF, sbuf, (j - 2) % 2)

            gwait(j)
            add_pos(j, hk)
            ostart(ck, j, hk)
        return carry

    lax.fori_loop(0, _GROUPS, group, 0)
    for j in range(_NBUF):
        owait(j)


@jax.jit
def kernel(sequence, token_table, pos_table):
    seq = sequence.astype(jnp.int32)
    pos_flat = pos_table.reshape(_L * _D)
    mesh = plsc.VectorSubcoreMesh(core_axis_name="c", subcore_axis_name="s")
    out = pl.kernel(
        _emb_body,
        out_type=jax.ShapeDtypeStruct((_B * _L, _D), jnp.float32),
        mesh=mesh,
        scratch_types=[
            pltpu.VMEM((_BW, _LW), jnp.int32),
            pltpu.VMEM((_LW * _D,), jnp.float32),
            pltpu.VMEM((_NBUF, _CW, _D), jnp.float32),
            [pltpu.SemaphoreType.DMA] * _NBUF,
            [pltpu.SemaphoreType.DMA] * _NBUF,
            pltpu.SemaphoreType.DMA,
        ],
    )(seq, token_table, pos_flat)
    return out.reshape(_B, _L, _D)


# speedup vs baseline: 1.0200x; 1.0200x over previous
import jax
import jax.numpy as jnp
from jax import lax
from jax.experimental import pallas as pl
from jax.experimental.pallas import tpu as pltpu
from jax.experimental.pallas import tpu_sc as plsc

_VOCAB = 100000
_D = 128
_L = 2048
_B = 32
_NC = 2
_NS = 16
_NW = _NC * _NS
_NHALF = 2
_BW = _B // _NHALF
_NCOL = _NW // _NHALF
_LW = _L // _NCOL
_LANES = 16
_NBUF = 4
_GROUPS = _BW // _NBUF


def _emb_body(seq_hbm, table_hbm, pos_hbm, out_hbm, idx_v, pos_v, tok_v,
              gsems, osems, psem):
    wid = lax.axis_index("s") * _NC + lax.axis_index("c")
    half = wid // _NCOL
    colblk = wid % _NCOL
    b0 = half * _BW
    l0 = colblk * _LW

    pltpu.sync_copy(seq_hbm.at[pl.ds(b0, _BW), pl.ds(l0, _LW)], idx_v)
    pos_copy = pltpu.async_copy(
        pos_hbm.at[pl.ds(l0 * _D, _LW * _D)], pos_v, psem)

    def gstart(lb, buf):
        pltpu.async_copy(table_hbm.at[idx_v.at[lb]], tok_v.at[buf], gsems[buf])

    def gwait(buf):
        pltpu.make_async_copy(
            table_hbm.at[pl.ds(0, _LW)], tok_v.at[buf], gsems[buf]).wait()

    def ostart(lb, buf):
        pltpu.async_copy(
            tok_v.at[buf],
            out_hbm.at[pl.ds((b0 + lb) * _L + l0, _LW)], osems[buf])

    def owait(buf):
        pltpu.make_async_copy(
            tok_v.at[buf], out_hbm.at[pl.ds(0, _LW)], osems[buf]).wait()

    def add_pos(buf):
        @pl.loop(0, _LW, unroll=1)
        def _(r):
            for c in range(_D // _LANES):
                plsc.addupdate(
                    tok_v.at[buf, r, pl.ds(c * _LANES, _LANES)],
                    pos_v[pl.ds(r * _D + c * _LANES, _LANES)],
                )

    for lb in range(_NBUF):
        gstart(lb, lb)
    pos_copy.wait()

    def group(i, carry):
        for j in range(_NBUF):
            lb = _NBUF * i + j
            s = lb - 2
            sbuf = (j - 2) % _NBUF

            @pl.when(jnp.logical_and(s >= 0, s + _NBUF < _BW))
            def _():
                owait(sbuf)
                gstart(s + _NBUF, sbuf)

            gwait(j)
            add_pos(j)
            ostart(lb, j)
        return carry

    lax.fori_loop(0, _GROUPS, group, 0)
    for j in range(_NBUF):
        owait(j)


@jax.jit
def kernel(sequence, token_table, pos_table):
    seq = sequence.astype(jnp.int32)
    pos_flat = pos_table.reshape(_L * _D)
    mesh = plsc.VectorSubcoreMesh(core_axis_name="c", subcore_axis_name="s")
    out = pl.kernel(
        _emb_body,
        out_type=jax.ShapeDtypeStruct((_B * _L, _D), jnp.float32),
        mesh=mesh,
        scratch_types=[
            pltpu.VMEM((_BW, _LW), jnp.int32),
            pltpu.VMEM((_LW * _D,), jnp.float32),
            pltpu.VMEM((_NBUF, _LW, _D), jnp.float32),
            [pltpu.SemaphoreType.DMA] * _NBUF,
            [pltpu.SemaphoreType.DMA] * _NBUF,
            pltpu.SemaphoreType.DMA,
        ],
    )(seq, token_table, pos_flat)
    return out.reshape(_B, _L, _D)
